# baseline (speedup 1.0000x reference)
import functools

import jax
import jax.numpy as jnp
from jax import lax
from jax.experimental import pallas as pl
from jax.experimental.pallas import tpu as pltpu
from jax.experimental.pallas import tpu_sc as plsc

_NUM_CORES = 2
_NUM_SUBCORES = 16
_NW = _NUM_CORES * _NUM_SUBCORES

_CHUNK = 128


def _gather_kernel(B: int, D: int):
    b_per_w = B // _NW
    n_chunks = b_per_w // _CHUNK
    mesh = plsc.VectorSubcoreMesh(core_axis_name="c", subcore_axis_name="s")

    @functools.partial(
        pl.kernel,
        mesh=mesh,
        compiler_params=pltpu.CompilerParams(use_tc_tiling_on_sc=False),
        out_type=jax.ShapeDtypeStruct((B, D), jnp.float32),
        scratch_types=[
            pltpu.VMEM((n_chunks, _CHUNK), jnp.int32),
            pltpu.VMEM((_CHUNK, D), jnp.float32),
            pltpu.SemaphoreType.DMA,
        ],
    )
    def k(table_hbm, idx_hbm, out_hbm, idx_v, rows_v, sem):
        wid = lax.axis_index("s") * _NUM_CORES + lax.axis_index("c")
        row0 = wid * n_chunks
        pltpu.sync_copy(idx_hbm.at[pl.ds(row0, n_chunks)], idx_v)
        base = wid * b_per_w

        def body(i, carry):
            pltpu.async_copy(table_hbm.at[idx_v.at[i]], rows_v, sem).wait()
            pltpu.sync_copy(rows_v, out_hbm.at[pl.ds(base + i * _CHUNK, _CHUNK)])
            return carry

        lax.fori_loop(0, n_chunks, body, 0, unroll=False)

    return k


def kernel(embedding, idx_mapping):
    B = idx_mapping.shape[0]
    D = embedding.shape[1]
    idx2d = idx_mapping.astype(jnp.int32).reshape(B // _CHUNK, _CHUNK)
    return _gather_kernel(B, D)(embedding, idx2d)


# speedup vs baseline: 1.0592x; 1.0592x over previous
import functools

import jax
import jax.numpy as jnp
from jax import lax
from jax.experimental import pallas as pl
from jax.experimental.pallas import tpu as pltpu
from jax.experimental.pallas import tpu_sc as plsc

_NUM_CORES = 2
_NUM_SUBCORES = 16
_NW = _NUM_CORES * _NUM_SUBCORES

_CHUNK = 512


def _gather_kernel(B: int, D: int):
    b_per_w = B // _NW
    n_chunks = b_per_w // _CHUNK
    mesh = plsc.VectorSubcoreMesh(core_axis_name="c", subcore_axis_name="s")

    @functools.partial(
        pl.kernel,
        mesh=mesh,
        compiler_params=pltpu.CompilerParams(use_tc_tiling_on_sc=False),
        out_type=jax.ShapeDtypeStruct((B, D), jnp.float32),
        scratch_types=[
            pltpu.VMEM((n_chunks, _CHUNK), jnp.int32),
            pltpu.VMEM((_CHUNK, D), jnp.float32),
            pltpu.SemaphoreType.DMA,
        ],
    )
    def k(table_hbm, idx_hbm, out_hbm, idx_v, rows_v, sem):
        wid = lax.axis_index("s") * _NUM_CORES + lax.axis_index("c")
        row0 = wid * n_chunks
        pltpu.sync_copy(idx_hbm.at[pl.ds(row0, n_chunks)], idx_v)
        base = wid * b_per_w

        def body(i, carry):
            pltpu.async_copy(table_hbm.at[idx_v.at[i]], rows_v, sem).wait()
            pltpu.sync_copy(rows_v, out_hbm.at[pl.ds(base + i * _CHUNK, _CHUNK)])
            return carry

        lax.fori_loop(0, n_chunks, body, 0, unroll=False)

    return k


def kernel(embedding, idx_mapping):
    B = idx_mapping.shape[0]
    D = embedding.shape[1]
    idx2d = idx_mapping.astype(jnp.int32).reshape(B // _CHUNK, _CHUNK)
    return _gather_kernel(B, D)(embedding, idx2d)


# speedup vs baseline: 1.0792x; 1.0189x over previous
import functools

import jax
import jax.numpy as jnp
from jax import lax
from jax.experimental import pallas as pl
from jax.experimental.pallas import tpu as pltpu
from jax.experimental.pallas import tpu_sc as plsc

_NUM_CORES = 2
_NUM_SUBCORES = 16
_NW = _NUM_CORES * _NUM_SUBCORES

_CHUNK = 256
_NSLOT = 4


def _gather_kernel(B: int, D: int):
    b_per_w = B // _NW
    n_sup = b_per_w // _CHUNK
    n_loop = n_sup // _NSLOT
    assert n_sup % _NSLOT == 0 and n_loop >= 2
    mesh = plsc.VectorSubcoreMesh(core_axis_name="c", subcore_axis_name="s")

    @functools.partial(
        pl.kernel,
        mesh=mesh,
        compiler_params=pltpu.CompilerParams(use_tc_tiling_on_sc=False),
        out_type=jax.ShapeDtypeStruct((B, D), jnp.float32),
        scratch_types=[pltpu.VMEM((n_sup, _CHUNK), jnp.int32)]
        + [pltpu.VMEM((_CHUNK, D), jnp.float32)] * _NSLOT
        + [pltpu.SemaphoreType.DMA] * (2 * _NSLOT),
    )
    def k(table_hbm, idx_hbm, out_hbm, idx_v, b0, b1, b2, b3,
          g0, g1, g2, g3, s0, s1, s2, s3):
        bufs = (b0, b1, b2, b3)
        gsems = (g0, g1, g2, g3)
        ssems = (s0, s1, s2, s3)
        wid = lax.axis_index("s") * _NUM_CORES + lax.axis_index("c")
        row0 = wid * n_sup
        base = wid * b_per_w
        pltpu.sync_copy(idx_hbm.at[pl.ds(row0, n_sup)], idx_v)

        def fire(g, r):
            pltpu.async_copy(table_hbm.at[idx_v.at[g]], bufs[r], gsems[r])

        def drain_gather(r):
            pltpu.make_async_copy(
                table_hbm.at[pl.ds(0, _CHUNK)], bufs[r], gsems[r]).wait()

        def store(g, r):
            pltpu.async_copy(
                bufs[r], out_hbm.at[pl.ds(base + g * _CHUNK, _CHUNK)], ssems[r])

        def drain_store(r):
            pltpu.make_async_copy(
                bufs[r], out_hbm.at[pl.ds(base, _CHUNK)], ssems[r]).wait()

        def visit(g, r, rn, guard_first):
            if not guard_first:
                drain_store(rn)
            fire(g + 1, rn)
            drain_gather(r)
            store(g, r)

        fire(0, 0)
        for g in range(_NSLOT):
            visit(g, g % _NSLOT, (g + 1) % _NSLOT, guard_first=g < _NSLOT - 1)

        def body(i, carry):
            gb = i * _NSLOT
            for j in range(_NSLOT):
                visit(gb + j, j, (j + 1) % _NSLOT, guard_first=False)
            return carry

        lax.fori_loop(1, n_loop - 1, body, 0, unroll=False)

        for g in range(n_sup - _NSLOT, n_sup):
            r = g % _NSLOT
            if g + 1 < n_sup:
                drain_store((g + 1) % _NSLOT)
                fire(g + 1, (g + 1) % _NSLOT)
            drain_gather(r)
            store(g, r)
        for r in range(_NSLOT):
            drain_store(r)

    return k


def kernel(embedding, idx_mapping):
    B = idx_mapping.shape[0]
    D = embedding.shape[1]
    idx2d = idx_mapping.astype(jnp.int32).reshape(B // _CHUNK, _CHUNK)
    return _gather_kernel(B, D)(embedding, idx2d)
